# baseline (speedup 1.0000x reference)
import jax
import jax.numpy as jnp
from jax import lax
from jax.experimental import pallas as pl
from jax.experimental.pallas import tpu as pltpu

N = 4096
C = 1024
E = 8
F = 4096
TS = F // E
BM = 256
NT = N // BM


def _ffn_body(x_ref, wg_ref, bg_ref, wu_ref, bu_ref, wd_ref, bd_ref,
              out_ref, gate_ref, xs_s, acc_s):
    e = pl.program_id(0)
    i = pl.program_id(1)
    rows = pl.ds(i * BM, BM)

    @pl.when(e == 0)
    def _():
        xs_s[rows, :] = x_ref[...]

    xb = xs_s[rows, :]
    logits = jnp.dot(xb, wg_ref[...],
                     preferred_element_type=jnp.float32) + bg_ref[...]
    lane = lax.broadcasted_iota(jnp.int32, logits.shape, 1)
    mx = jnp.max(logits, axis=-1, keepdims=True)
    idx = jnp.min(jnp.where(logits == mx, lane, E), axis=-1,
                  keepdims=True)
    gate_ref[...] = (lane == idx).astype(jnp.float32)

    h = jnp.dot(xb, wu_ref[...],
                preferred_element_type=jnp.float32) + bu_ref[...]
    h = jnp.where(idx == e, jnp.maximum(h, 0.0), 0.0)
    contrib = jnp.dot(h, wd_ref[...], preferred_element_type=jnp.float32)

    @pl.when(e == 0)
    def _():
        acc_s[rows, :] = contrib + bd_ref[...]

    @pl.when(e > 0)
    def _():
        acc_s[rows, :] = acc_s[rows, :] + contrib

    @pl.when(e == E - 1)
    def _():
        out_ref[...] = acc_s[rows, :]


def kernel(x, W_gate, b_gate, W_up, b_up, W_down, b_down):
    B, T, Cx = x.shape
    assert (B * T, Cx, W_gate.shape[1], W_up.shape[1]) == (N, C, E, F)
    x_f = x.reshape(N, C)

    out, gate = pl.pallas_call(
        _ffn_body,
        grid=(E, NT),
        in_specs=[
            pl.BlockSpec((BM, C), lambda e, i: (jnp.where(e == 0, i, 0), 0)),
            pl.BlockSpec((C, E), lambda e, i: (0, 0)),
            pl.BlockSpec((1, E), lambda e, i: (0, 0)),
            pl.BlockSpec((C, TS), lambda e, i: (0, e)),
            pl.BlockSpec((1, TS), lambda e, i: (0, e)),
            pl.BlockSpec((TS, C), lambda e, i: (e, 0)),
            pl.BlockSpec((1, C), lambda e, i: (0, 0)),
        ],
        out_specs=[
            pl.BlockSpec((BM, C),
                         lambda e, i: (jnp.where(e == E - 1, i, 0), 0)),
            pl.BlockSpec((BM, E), lambda e, i: (i, 0)),
        ],
        out_shape=[
            jax.ShapeDtypeStruct((N, C), jnp.float32),
            jax.ShapeDtypeStruct((N, E), jnp.float32),
        ],
        scratch_shapes=[
            pltpu.VMEM((N, C), jnp.float32),
            pltpu.VMEM((N, C), jnp.float32),
        ],
        compiler_params=pltpu.CompilerParams(
            vmem_limit_bytes=112 * 1024 * 1024,
        ),
    )(x_f, W_gate, b_gate.reshape(1, E), W_up, b_up.reshape(1, F),
      W_down, b_down.reshape(1, C))
    return out.reshape(B, T, C), gate.reshape(B, T, E)


# speedup vs baseline: 2.1111x; 2.1111x over previous
import functools
import jax
import jax.numpy as jnp
from jax import lax
from jax.experimental import pallas as pl
from jax.experimental.pallas import tpu as pltpu


def _ffn_body(x_ref, wg_ref, bg_ref, wu0_ref, wu1_ref, bu_ref,
              wd0_ref, wd1_ref, bd_ref, out_ref, gate_ref, *, ts):
    xb = x_ref[...]
    logits = jnp.dot(xb, wg_ref[...],
                     preferred_element_type=jnp.float32) + bg_ref[...]
    lane = lax.broadcasted_iota(jnp.int32, logits.shape, 1)
    mx = jnp.max(logits, axis=-1, keepdims=True)
    num_e = logits.shape[-1]
    idx = jnp.min(jnp.where(logits == mx, lane, num_e), axis=-1,
                  keepdims=True)
    onehot = (lane == idx).astype(jnp.float32)
    gate_ref[...] = onehot
    h = jnp.concatenate(
        [jnp.dot(xb, wu0_ref[...], preferred_element_type=jnp.float32),
         jnp.dot(xb, wu1_ref[...], preferred_element_type=jnp.float32)],
        axis=1) + bu_ref[...]
    tile_of_feat = lax.broadcasted_iota(jnp.int32, h.shape, 1) // ts
    h = jnp.where(tile_of_feat == idx, h, 0.0)
    h = jnp.maximum(h, 0.0)
    half = h.shape[1] // 2
    out_ref[...] = (
        jnp.dot(h[:, :half], wd0_ref[...], preferred_element_type=jnp.float32)
        + jnp.dot(h[:, half:], wd1_ref[...],
                  preferred_element_type=jnp.float32)
        + bd_ref[...])


def kernel(x, W_gate, b_gate, W_up, b_up, W_down, b_down):
    B, T, C = x.shape
    N = B * T
    E = W_gate.shape[1]
    F = W_up.shape[1]
    TS = F // E
    FH = F // 2
    x_f = x.reshape(N, C)
    BM = min(256, N)

    body = functools.partial(_ffn_body, ts=TS)
    out, gate = pl.pallas_call(
        body,
        grid=(N // BM,),
        in_specs=[
            pl.BlockSpec((BM, C), lambda i: (i, 0)),
            pl.BlockSpec((C, E), lambda i: (0, 0)),
            pl.BlockSpec((1, E), lambda i: (0, 0)),
            pl.BlockSpec((C, FH), lambda i: (0, 0)),
            pl.BlockSpec((C, FH), lambda i: (0, 1)),
            pl.BlockSpec((1, F), lambda i: (0, 0)),
            pl.BlockSpec((FH, C), lambda i: (0, 0)),
            pl.BlockSpec((FH, C), lambda i: (1, 0)),
            pl.BlockSpec((1, C), lambda i: (0, 0)),
        ],
        out_specs=[
            pl.BlockSpec((BM, C), lambda i: (i, 0)),
            pl.BlockSpec((BM, E), lambda i: (i, 0)),
        ],
        out_shape=[
            jax.ShapeDtypeStruct((N, C), jnp.float32),
            jax.ShapeDtypeStruct((N, E), jnp.float32),
        ],
        compiler_params=pltpu.CompilerParams(
            vmem_limit_bytes=112 * 1024 * 1024,
        ),
    )(x_f, W_gate, b_gate.reshape(1, E), W_up, W_up,
      b_up.reshape(1, F), W_down, W_down,
      b_down.reshape(1, C))
    return out.reshape(B, T, C), gate.reshape(B, T, E)


# speedup vs baseline: 2.1556x; 1.0211x over previous
import functools
import jax
import jax.numpy as jnp
from jax import lax
from jax.experimental import pallas as pl
from jax.experimental.pallas import tpu as pltpu


def _ffn_body(x_ref, wg_ref, bg_ref, wu0_ref, wu1_ref, bu_ref,
              wd0_ref, wd1_ref, bd_ref, out_ref, gate_ref, *, ts):
    xb = x_ref[...]
    logits = jnp.dot(xb, wg_ref[...],
                     preferred_element_type=jnp.float32) + bg_ref[...]
    lane = lax.broadcasted_iota(jnp.int32, logits.shape, 1)
    mx = jnp.max(logits, axis=-1, keepdims=True)
    num_e = logits.shape[-1]
    idx = jnp.min(jnp.where(logits == mx, lane, num_e), axis=-1,
                  keepdims=True)
    onehot = (lane == idx).astype(jnp.float32)
    gate_ref[...] = onehot
    h = jnp.concatenate(
        [jnp.dot(xb, wu0_ref[...], preferred_element_type=jnp.float32),
         jnp.dot(xb, wu1_ref[...], preferred_element_type=jnp.float32)],
        axis=1) + bu_ref[...]
    tile_of_feat = lax.broadcasted_iota(jnp.int32, h.shape, 1) // ts
    h = jnp.where(tile_of_feat == idx, h, 0.0)
    h = jnp.maximum(h, 0.0)
    half = h.shape[1] // 2
    out_ref[...] = (
        jnp.dot(h[:, :half], wd0_ref[...], preferred_element_type=jnp.float32)
        + jnp.dot(h[:, half:], wd1_ref[...],
                  preferred_element_type=jnp.float32)
        + bd_ref[...])


def kernel(x, W_gate, b_gate, W_up, b_up, W_down, b_down):
    B, T, C = x.shape
    N = B * T
    E = W_gate.shape[1]
    F = W_up.shape[1]
    TS = F // E
    FH = F // 2
    x_f = x.reshape(N, C)
    BM = min(512, N)

    body = functools.partial(_ffn_body, ts=TS)
    out, gate = pl.pallas_call(
        body,
        grid=(N // BM,),
        in_specs=[
            pl.BlockSpec((BM, C), lambda i: (i, 0)),
            pl.BlockSpec((C, E), lambda i: (0, 0)),
            pl.BlockSpec((1, E), lambda i: (0, 0)),
            pl.BlockSpec((C, FH), lambda i: (0, 0)),
            pl.BlockSpec((C, FH), lambda i: (0, 1)),
            pl.BlockSpec((1, F), lambda i: (0, 0)),
            pl.BlockSpec((FH, C), lambda i: (0, 0)),
            pl.BlockSpec((FH, C), lambda i: (1, 0)),
            pl.BlockSpec((1, C), lambda i: (0, 0)),
        ],
        out_specs=[
            pl.BlockSpec((BM, C), lambda i: (i, 0)),
            pl.BlockSpec((BM, E), lambda i: (i, 0)),
        ],
        out_shape=[
            jax.ShapeDtypeStruct((N, C), jnp.float32),
            jax.ShapeDtypeStruct((N, E), jnp.float32),
        ],
        compiler_params=pltpu.CompilerParams(
            vmem_limit_bytes=112 * 1024 * 1024,
        ),
    )(x_f, W_gate, b_gate.reshape(1, E), W_up, W_up,
      b_up.reshape(1, F), W_down, W_down,
      b_down.reshape(1, C))
    return out.reshape(B, T, C), gate.reshape(B, T, E)
